# baseline (speedup 1.0000x reference)
import functools

import jax
import jax.numpy as jnp
from jax import lax
from jax.experimental import pallas as pl
from jax.experimental.pallas import tpu as pltpu
from jax.experimental.pallas import tpu_sc as plsc

N = 10000
E = 320000
NC, NS = 2, 16
NW = NC * NS
EPW = E // NW
K = 40
NWIN = EPW // K
RPT = 1000
RPT2 = 2000

_MESH = plsc.VectorSubcoreMesh(core_axis_name="c", subcore_axis_name="s")


def _sc_degrees(src3, dstn3, ew, ones_k, z2n):
    @functools.partial(
        pl.kernel,
        mesh=_MESH,
        compiler_params=pltpu.CompilerParams(
            needs_layout_passes=False, use_tc_tiling_on_sc=False),
        out_type=jax.ShapeDtypeStruct((NC * 2 * 2 * N,), jnp.float32),
        scratch_types=[
            pltpu.VMEM((NWIN, K), jnp.int32),
            pltpu.VMEM((NWIN, K), jnp.int32),
            pltpu.VMEM((EPW,), jnp.float32),
            pltpu.VMEM((K,), jnp.float32),
            pltpu.VMEM((RPT2,), jnp.float32),
            pltpu.SemaphoreType.DMA,
            pltpu.VMEM_SHARED((2 * N,), jnp.float32),
            pltpu.VMEM_SHARED((2 * N,), jnp.float32),
        ],
    )
    def k(src_hbm, dstn_hbm, ew_hbm, ones_hbm, z_hbm, out_hbm,
          srcv, dstnv, ewv, onesv, buf, sem, wacc, dacc):
        cid = lax.axis_index("c")
        sid = lax.axis_index("s")
        wid = sid * NC + cid
        pltpu.sync_copy(src_hbm.at[wid], srcv)
        pltpu.sync_copy(dstn_hbm.at[wid], dstnv)
        pltpu.sync_copy(ew_hbm.at[pl.ds(wid * EPW, EPW)], ewv)
        pltpu.sync_copy(ones_hbm, onesv)
        @pl.when(sid < 10)
        def _():
            pltpu.sync_copy(z_hbm.at[pl.ds(sid * RPT2, RPT2)], buf)
            pltpu.sync_copy(buf, wacc.at[pl.ds(sid * RPT2, RPT2)])
            pltpu.sync_copy(buf, dacc.at[pl.ds(sid * RPT2, RPT2)])
        plsc.subcore_barrier()

        def win(w, carry):
            ews = ewv.at[pl.ds(w * K, K)]
            pltpu.async_copy(ews, wacc.at[srcv.at[w]], sem, add=True)
            pltpu.async_copy(ews, wacc.at[dstnv.at[w]], sem, add=True)
            pltpu.async_copy(onesv, dacc.at[srcv.at[w]], sem, add=True)
            pltpu.async_copy(onesv, dacc.at[dstnv.at[w]], sem, add=True)
            return carry

        lax.fori_loop(0, NWIN, win, 0)

        def drain(w, carry):
            pltpu.make_async_copy(
                ewv.at[pl.ds(0, K)], wacc.at[srcv.at[0]], sem).wait()
            return carry

        lax.fori_loop(0, 4 * NWIN, drain, 0)
        plsc.subcore_barrier()

        @pl.when(sid < 10)
        def _():
            pltpu.sync_copy(wacc.at[pl.ds(sid * RPT2, RPT2)], buf)
            pltpu.sync_copy(
                buf, out_hbm.at[pl.ds(cid * 4 * N + sid * RPT2, RPT2)])
            pltpu.sync_copy(dacc.at[pl.ds(sid * RPT2, RPT2)], buf)
            pltpu.sync_copy(
                buf, out_hbm.at[pl.ds(cid * 4 * N + 2 * N + sid * RPT2, RPT2)])

    return k(src3, dstn3, ew, ones_k, z2n)


def _sc_layer(gt, src, dst3, m, zeros, d):
    @functools.partial(
        pl.kernel,
        mesh=_MESH,
        compiler_params=pltpu.CompilerParams(
            needs_layout_passes=False, use_tc_tiling_on_sc=False),
        out_type=jax.ShapeDtypeStruct((NC * N, d), jnp.float32),
        scratch_types=[
            pltpu.VMEM((EPW,), jnp.int32),
            pltpu.VMEM((NWIN, K), jnp.int32),
            pltpu.VMEM((EPW,), jnp.float32),
            pltpu.VMEM((K, d), jnp.float32),
            pltpu.VMEM((K, d), jnp.float32),
            pltpu.VMEM((K, d), jnp.float32),
            pltpu.SemaphoreType.DMA,
            pltpu.SemaphoreType.DMA,
            pltpu.SemaphoreType.DMA,
            pltpu.SemaphoreType.DMA,
            pltpu.SemaphoreType.DMA,
            pltpu.SemaphoreType.DMA,
            pltpu.VMEM_SHARED((N, d), jnp.float32),
        ],
    )
    def k(gt_hbm, src_hbm, dst_hbm, m_hbm, z_hbm, out_hbm,
          srcv, dstv, mv, rows0, rows1, rows2,
          gsem0, gsem1, gsem2, ssem0, ssem1, ssem2, acc):
        cid = lax.axis_index("c")
        sid = lax.axis_index("s")
        wid = sid * NC + cid
        pltpu.sync_copy(src_hbm.at[pl.ds(wid * EPW, EPW)], srcv)
        pltpu.sync_copy(dst_hbm.at[wid], dstv)
        pltpu.sync_copy(m_hbm.at[pl.ds(wid * EPW, EPW)], mv)
        @pl.when(sid < 10)
        def _():
            pltpu.sync_copy(z_hbm.at[pl.ds(sid * RPT, RPT)],
                            acc.at[pl.ds(sid * RPT, RPT)])
        plsc.subcore_barrier()

        def start(w, rows, gsem):
            pltpu.async_copy(gt_hbm.at[srcv.at[pl.ds(w * K, K)]], rows, gsem)

        def proc(w, rows, gsem, ssem):
            pltpu.make_async_copy(
                gt_hbm.at[srcv.at[pl.ds(w * K, K)]], rows, gsem).wait()

            @plsc.parallel_loop(0, K, unroll=4)
            def row(r):
                bidx = jnp.full((16,), w * K + r, jnp.int32)
                wvec = plsc.load_gather(mv, [bidx])
                for j in range(d // 16):
                    sl = pl.ds(j * 16, 16)
                    rows[r, sl] = rows[r, sl] * wvec
            pltpu.async_copy(rows, acc.at[dstv.at[w]], ssem, add=True)

        def wait_scat(w, rows, ssem):
            pltpu.make_async_copy(rows, acc.at[dstv.at[w]], ssem).wait()

        start(0, rows0, gsem0)
        start(1, rows1, gsem1)
        proc(0, rows0, gsem0, ssem0)
        start(2, rows2, gsem2)

        def tri(t, carry):
            w = 3 * t + 1
            proc(w, rows1, gsem1, ssem1)
            wait_scat(w - 1, rows0, ssem0)
            start(w + 2, rows0, gsem0)
            proc(w + 1, rows2, gsem2, ssem2)
            wait_scat(w, rows1, ssem1)
            start(w + 3, rows1, gsem1)
            proc(w + 2, rows0, gsem0, ssem0)
            wait_scat(w + 1, rows2, ssem2)
            start(w + 4, rows2, gsem2)
            return carry

        lax.fori_loop(0, (NWIN - 7) // 3, tri, 0)
        proc(NWIN - 6, rows1, gsem1, ssem1)
        wait_scat(NWIN - 7, rows0, ssem0)
        start(NWIN - 4, rows0, gsem0)
        proc(NWIN - 5, rows2, gsem2, ssem2)
        wait_scat(NWIN - 6, rows1, ssem1)
        start(NWIN - 3, rows1, gsem1)
        proc(NWIN - 4, rows0, gsem0, ssem0)
        wait_scat(NWIN - 5, rows2, ssem2)
        start(NWIN - 2, rows2, gsem2)
        proc(NWIN - 3, rows1, gsem1, ssem1)
        wait_scat(NWIN - 4, rows0, ssem0)
        start(NWIN - 1, rows0, gsem0)
        proc(NWIN - 2, rows2, gsem2, ssem2)
        proc(NWIN - 1, rows0, gsem0, ssem0)
        wait_scat(NWIN - 3, rows1, ssem1)
        wait_scat(NWIN - 2, rows2, ssem2)
        wait_scat(NWIN - 1, rows0, ssem0)
        plsc.subcore_barrier()

        @pl.when(sid < 10)
        def _():
            pltpu.sync_copy(acc.at[pl.ds(sid * RPT, RPT)],
                            out_hbm.at[pl.ds(cid * N + sid * RPT, RPT)])

    return k(gt, src, dst3, m, zeros)


def _tc_first(d4, feats, w0):
    def body(d_ref, x_ref, w_ref, s_ref, t_ref, gt_ref):
        wdeg = d_ref[0, :] + d_ref[2, :]
        deg = d_ref[1, :] + d_ref[3, :]
        wdeg = jnp.where(wdeg <= 0.0, 1.0, wdeg)
        deg = jnp.maximum(deg, 1.0)
        st = lax.rsqrt(wdeg) * lax.rsqrt(deg)
        s = st[:N].reshape(N, 1)
        t = st[N:].reshape(N, 1)
        s_ref[...] = s
        t_ref[...] = t
        gt_ref[...] = jnp.dot(x_ref[...] * s, w_ref[...],
                              preferred_element_type=jnp.float32)

    return pl.pallas_call(
        body,
        out_shape=(
            jax.ShapeDtypeStruct((N, 1), jnp.float32),
            jax.ShapeDtypeStruct((N, 1), jnp.float32),
            jax.ShapeDtypeStruct((N, 128), jnp.float32),
        ),
    )(d4, feats, w0)


def _tc_mid(pp, b, w, scol, tcol):
    dn = w.shape[1]

    def body(p_ref, b_ref, w_ref, s_ref, t_ref, gt_ref):
        p = p_ref[pl.ds(0, N), :] + p_ref[pl.ds(N, N), :]
        x = jnp.maximum(p * t_ref[...] + b_ref[...], 0.0)
        gt_ref[...] = jnp.dot(x * s_ref[...], w_ref[...],
                              preferred_element_type=jnp.float32)

    return pl.pallas_call(
        body,
        out_shape=jax.ShapeDtypeStruct((N, dn), jnp.float32),
    )(pp, b, w, scol, tcol)


def _tc_final(pp, b, tcol):
    d = b.shape[1]

    def body(p_ref, b_ref, t_ref, o_ref):
        p = p_ref[pl.ds(0, N), pl.ds(0, d)] + p_ref[pl.ds(N, N), pl.ds(0, d)]
        o_ref[...] = p * t_ref[...] + b_ref[...]

    return pl.pallas_call(
        body,
        out_shape=jax.ShapeDtypeStruct((N, d), jnp.float32),
    )(pp, b, tcol)


def kernel(features, edge_index, edge_weight, W0, b0, W1, b1, W2, b2):
    src = edge_index[0]
    dst = edge_index[1]
    src3 = src.reshape(NW, NWIN, K)
    dst3 = dst.reshape(NW, NWIN, K)
    dstn3 = (dst + N).reshape(NW, NWIN, K)
    ones_k = jnp.ones((K,), jnp.float32)
    z2n = jnp.zeros((2 * N,), jnp.float32)
    z128 = jnp.zeros((N, 128), jnp.float32)
    W2p = jnp.pad(W2, ((0, 0), (0, 128 - W2.shape[1])))

    dtab = _sc_degrees(src3, dstn3, edge_weight, ones_k, z2n)
    scol, tcol, gt0 = _tc_first(dtab.reshape(4, 2 * N), features, W0)
    p0 = _sc_layer(gt0, src, dst3, edge_weight, z128, 128)
    gt1 = _tc_mid(p0, b0.reshape(1, 128), W1, scol, tcol)
    p1 = _sc_layer(gt1, src, dst3, edge_weight, z128, 128)
    gt2 = _tc_mid(p1, b1.reshape(1, 128), W2p, scol, tcol)
    p2 = _sc_layer(gt2, src, dst3, edge_weight, z128, 128)
    return _tc_final(p2, b2.reshape(1, 64), tcol)


# speedup vs baseline: 1.0932x; 1.0932x over previous
import functools

import jax
import jax.numpy as jnp
from jax import lax
from jax.experimental import pallas as pl
from jax.experimental.pallas import tpu as pltpu
from jax.experimental.pallas import tpu_sc as plsc

N = 10000
E = 320000
NC, NS = 2, 16
NW = NC * NS
EPW = E // NW
K = 80
NWIN = EPW // K
RPT = 1000
RPT2 = 2000

_MESH = plsc.VectorSubcoreMesh(core_axis_name="c", subcore_axis_name="s")


def _sc_degrees(src3, dstn3, ew, ones_k, z2n):
    @functools.partial(
        pl.kernel,
        mesh=_MESH,
        compiler_params=pltpu.CompilerParams(
            needs_layout_passes=False, use_tc_tiling_on_sc=False),
        out_type=jax.ShapeDtypeStruct((NC * 2 * 2 * N,), jnp.float32),
        scratch_types=[
            pltpu.VMEM((NWIN, K), jnp.int32),
            pltpu.VMEM((NWIN, K), jnp.int32),
            pltpu.VMEM((EPW,), jnp.float32),
            pltpu.VMEM((K,), jnp.float32),
            pltpu.VMEM((RPT2,), jnp.float32),
            pltpu.SemaphoreType.DMA,
            pltpu.VMEM_SHARED((2 * N,), jnp.float32),
            pltpu.VMEM_SHARED((2 * N,), jnp.float32),
        ],
    )
    def k(src_hbm, dstn_hbm, ew_hbm, ones_hbm, z_hbm, out_hbm,
          srcv, dstnv, ewv, onesv, buf, sem, wacc, dacc):
        cid = lax.axis_index("c")
        sid = lax.axis_index("s")
        wid = sid * NC + cid
        pltpu.sync_copy(src_hbm.at[wid], srcv)
        pltpu.sync_copy(dstn_hbm.at[wid], dstnv)
        pltpu.sync_copy(ew_hbm.at[pl.ds(wid * EPW, EPW)], ewv)
        pltpu.sync_copy(ones_hbm, onesv)
        @pl.when(sid < 10)
        def _():
            pltpu.sync_copy(z_hbm.at[pl.ds(sid * RPT2, RPT2)], buf)
            pltpu.sync_copy(buf, wacc.at[pl.ds(sid * RPT2, RPT2)])
            pltpu.sync_copy(buf, dacc.at[pl.ds(sid * RPT2, RPT2)])
        plsc.subcore_barrier()

        def win(w, carry):
            ews = ewv.at[pl.ds(w * K, K)]
            pltpu.async_copy(ews, wacc.at[srcv.at[w]], sem, add=True)
            pltpu.async_copy(ews, wacc.at[dstnv.at[w]], sem, add=True)
            pltpu.async_copy(onesv, dacc.at[srcv.at[w]], sem, add=True)
            pltpu.async_copy(onesv, dacc.at[dstnv.at[w]], sem, add=True)
            return carry

        lax.fori_loop(0, NWIN, win, 0)

        def drain(w, carry):
            pltpu.make_async_copy(
                ewv.at[pl.ds(0, K)], wacc.at[srcv.at[0]], sem).wait()
            return carry

        lax.fori_loop(0, 4 * NWIN, drain, 0)
        plsc.subcore_barrier()

        @pl.when(sid < 10)
        def _():
            pltpu.sync_copy(wacc.at[pl.ds(sid * RPT2, RPT2)], buf)
            pltpu.sync_copy(
                buf, out_hbm.at[pl.ds(cid * 4 * N + sid * RPT2, RPT2)])
            pltpu.sync_copy(dacc.at[pl.ds(sid * RPT2, RPT2)], buf)
            pltpu.sync_copy(
                buf, out_hbm.at[pl.ds(cid * 4 * N + 2 * N + sid * RPT2, RPT2)])

    return k(src3, dstn3, ew, ones_k, z2n)


def _sc_layer(gt, src, dst3, m, zeros, d):
    @functools.partial(
        pl.kernel,
        mesh=_MESH,
        compiler_params=pltpu.CompilerParams(
            needs_layout_passes=False, use_tc_tiling_on_sc=False),
        out_type=jax.ShapeDtypeStruct((NC * N, d), jnp.float32),
        scratch_types=[
            pltpu.VMEM((EPW,), jnp.int32),
            pltpu.VMEM((NWIN, K), jnp.int32),
            pltpu.VMEM((EPW,), jnp.float32),
            pltpu.VMEM((K, d), jnp.float32),
            pltpu.VMEM((K, d), jnp.float32),
            pltpu.SemaphoreType.DMA,
            pltpu.SemaphoreType.DMA,
            pltpu.VMEM_SHARED((N, d), jnp.float32),
        ],
    )
    def k(gt_hbm, src_hbm, dst_hbm, m_hbm, z_hbm, out_hbm,
          srcv, dstv, mv, rows0, rows1, gsem0, gsem1, acc):
        cid = lax.axis_index("c")
        sid = lax.axis_index("s")
        wid = sid * NC + cid
        pltpu.sync_copy(src_hbm.at[pl.ds(wid * EPW, EPW)], srcv)
        pltpu.sync_copy(dst_hbm.at[wid], dstv)
        pltpu.sync_copy(m_hbm.at[pl.ds(wid * EPW, EPW)], mv)
        @pl.when(sid < 10)
        def _():
            pltpu.sync_copy(z_hbm.at[pl.ds(sid * RPT, RPT)],
                            acc.at[pl.ds(sid * RPT, RPT)])
        plsc.subcore_barrier()

        def start(w, rows, gsem):
            pltpu.async_copy(gt_hbm.at[srcv.at[pl.ds(w * K, K)]], rows, gsem)

        def finish(w, rows, gsem):
            pltpu.make_async_copy(
                gt_hbm.at[srcv.at[pl.ds(w * K, K)]], rows, gsem).wait()

            @plsc.parallel_loop(0, K, unroll=4)
            def row(r):
                bidx = jnp.full((16,), w * K + r, jnp.int32)
                wvec = plsc.load_gather(mv, [bidx])
                for j in range(d // 16):
                    sl = pl.ds(j * 16, 16)
                    rows[r, sl] = rows[r, sl] * wvec
            pltpu.sync_copy(rows, acc.at[dstv.at[w]], add=True)

        start(0, rows0, gsem0)

        def pair(g, carry):
            w = 2 * g
            start(w + 1, rows1, gsem1)
            finish(w, rows0, gsem0)
            start(w + 2, rows0, gsem0)
            finish(w + 1, rows1, gsem1)
            return carry

        lax.fori_loop(0, (NWIN - 1) // 2, pair, 0)
        finish(NWIN - 1, rows0, gsem0)
        plsc.subcore_barrier()

        @pl.when(sid < 10)
        def _():
            pltpu.sync_copy(acc.at[pl.ds(sid * RPT, RPT)],
                            out_hbm.at[pl.ds(cid * N + sid * RPT, RPT)])

    return k(gt, src, dst3, m, zeros)


def _tc_first(d4, feats, w0):
    def body(d_ref, x_ref, w_ref, s_ref, t_ref, gt_ref):
        wdeg = d_ref[0, :] + d_ref[2, :]
        deg = d_ref[1, :] + d_ref[3, :]
        wdeg = jnp.where(wdeg <= 0.0, 1.0, wdeg)
        deg = jnp.maximum(deg, 1.0)
        st = lax.rsqrt(wdeg) * lax.rsqrt(deg)
        s = st[:N].reshape(N, 1)
        t = st[N:].reshape(N, 1)
        s_ref[...] = s
        t_ref[...] = t
        gt_ref[...] = jnp.dot(x_ref[...] * s, w_ref[...],
                              preferred_element_type=jnp.float32)

    return pl.pallas_call(
        body,
        out_shape=(
            jax.ShapeDtypeStruct((N, 1), jnp.float32),
            jax.ShapeDtypeStruct((N, 1), jnp.float32),
            jax.ShapeDtypeStruct((N, 128), jnp.float32),
        ),
    )(d4, feats, w0)


def _tc_mid(pp, b, w, scol, tcol):
    dn = w.shape[1]

    def body(p_ref, b_ref, w_ref, s_ref, t_ref, gt_ref):
        p = p_ref[pl.ds(0, N), :] + p_ref[pl.ds(N, N), :]
        x = jnp.maximum(p * t_ref[...] + b_ref[...], 0.0)
        gt_ref[...] = jnp.dot(x * s_ref[...], w_ref[...],
                              preferred_element_type=jnp.float32)

    return pl.pallas_call(
        body,
        out_shape=jax.ShapeDtypeStruct((N, dn), jnp.float32),
    )(pp, b, w, scol, tcol)


def _tc_final(pp, b, tcol):
    d = b.shape[1]

    def body(p_ref, b_ref, t_ref, o_ref):
        p = p_ref[pl.ds(0, N), pl.ds(0, d)] + p_ref[pl.ds(N, N), pl.ds(0, d)]
        o_ref[...] = p * t_ref[...] + b_ref[...]

    return pl.pallas_call(
        body,
        out_shape=jax.ShapeDtypeStruct((N, d), jnp.float32),
    )(pp, b, tcol)


def kernel(features, edge_index, edge_weight, W0, b0, W1, b1, W2, b2):
    src = edge_index[0]
    dst = edge_index[1]
    src3 = src.reshape(NW, NWIN, K)
    dst3 = dst.reshape(NW, NWIN, K)
    dstn3 = (dst + N).reshape(NW, NWIN, K)
    ones_k = jnp.ones((K,), jnp.float32)
    z2n = jnp.zeros((2 * N,), jnp.float32)
    z128 = jnp.zeros((N, 128), jnp.float32)
    W2p = jnp.pad(W2, ((0, 0), (0, 128 - W2.shape[1])))

    dtab = _sc_degrees(src3, dstn3, edge_weight, ones_k, z2n)
    scol, tcol, gt0 = _tc_first(dtab.reshape(4, 2 * N), features, W0)
    p0 = _sc_layer(gt0, src, dst3, edge_weight, z128, 128)
    gt1 = _tc_mid(p0, b0.reshape(1, 128), W1, scol, tcol)
    p1 = _sc_layer(gt1, src, dst3, edge_weight, z128, 128)
    gt2 = _tc_mid(p1, b1.reshape(1, 128), W2p, scol, tcol)
    p2 = _sc_layer(gt2, src, dst3, edge_weight, z128, 128)
    return _tc_final(p2, b2.reshape(1, 64), tcol)
